# Initial kernel scaffold; baseline (speedup 1.0000x reference)
import jax
import jax.numpy as jnp
from jax.experimental import pallas as pl


def kernel(x, gate_w, W_in, W_gate, W_out, norm_w):
    raise NotImplementedError("write your pallas kernel here")


# speedup vs baseline: 62.1775x; 62.1775x over previous
import functools

import jax
import jax.numpy as jnp
from jax.experimental import pallas as pl
from jax.experimental.pallas import tpu as pltpu

NUM_EXPERTS = 8
TOP_K = 2
DIM_MODEL = 768
DIM_EXPERT = 128
S = 2048
EPS = 1e-6

_BLK = 512


def _moe_body(x_ref, gw_ref, win_ref, wgate_ref, wout_ref, nw_ref, o_ref):
    x = x_ref[...]
    nw = nw_ref[...]
    y = x * jax.lax.rsqrt(jnp.mean(x * x, axis=1, keepdims=True) + EPS) * nw

    logits = jax.lax.dot_general(
        y, gw_ref[...], (((1,), (1,)), ((), ())),
        preferred_element_type=jnp.float32)
    ii = jax.lax.broadcasted_iota(jnp.int32, logits.shape, 1)
    m1 = jnp.max(logits, axis=1, keepdims=True)
    i1 = jnp.min(jnp.where(logits == m1, ii, NUM_EXPERTS), axis=1, keepdims=True)
    masked = jnp.where(ii == i1, -jnp.inf, logits)
    m2 = jnp.max(masked, axis=1, keepdims=True)
    i2 = jnp.min(jnp.where(masked == m2, ii, NUM_EXPERTS), axis=1, keepdims=True)
    w1 = 1.0 / (1.0 + jnp.exp(m2 - m1))
    w2 = 1.0 - w1

    win = win_ref[...].reshape(NUM_EXPERTS * DIM_EXPERT, DIM_MODEL)
    wg = wgate_ref[...].reshape(NUM_EXPERTS * DIM_EXPERT, DIM_MODEL)
    a = jax.lax.dot_general(y, win, (((1,), (1,)), ((), ())),
                            preferred_element_type=jnp.float32)
    b = jax.lax.dot_general(y, wg, (((1,), (1,)), ((), ())),
                            preferred_element_type=jnp.float32)
    h = a * (1.0 / (1.0 + jnp.exp(-a))) * b

    acc = x
    for e in range(NUM_EXPERTS):
        coeff = (jnp.where(i1 == e, w1, 0.0) + jnp.where(i2 == e, w2, 0.0))
        he = h[:, e * DIM_EXPERT:(e + 1) * DIM_EXPERT] * coeff
        acc = acc + jax.lax.dot_general(
            he, wout_ref[e], (((1,), (1,)), ((), ())),
            preferred_element_type=jnp.float32)
    o_ref[...] = acc


@jax.jit
def kernel(x, gate_w, W_in, W_gate, W_out, norm_w):
    b, s, m = x.shape
    x2 = x.reshape(s, m)
    nw = norm_w.reshape(1, m)
    grid = (s // _BLK,)
    out = pl.pallas_call(
        _moe_body,
        grid=grid,
        in_specs=[
            pl.BlockSpec((_BLK, m), lambda i: (i, 0)),
            pl.BlockSpec(gate_w.shape, lambda i: (0, 0)),
            pl.BlockSpec(W_in.shape, lambda i: (0, 0, 0)),
            pl.BlockSpec(W_gate.shape, lambda i: (0, 0, 0)),
            pl.BlockSpec(W_out.shape, lambda i: (0, 0, 0)),
            pl.BlockSpec((1, m), lambda i: (0, 0)),
        ],
        out_specs=pl.BlockSpec((_BLK, m), lambda i: (i, 0)),
        out_shape=jax.ShapeDtypeStruct((s, m), jnp.float32),
    )(x2, gate_w, W_in, W_gate, W_out, nw)
    return out.reshape(b, s, m)
